# baseline (speedup 1.0000x reference)
import functools

import jax
import jax.numpy as jnp
from jax.experimental import pallas as pl
from jax.experimental.pallas import tpu as pltpu

_MARGIN = 1.0
_R = 256
_C = 256
_KAUG = 136


def _triplet_block_kernel(nc, erow_ref, eallt_ref, labr_ref, labc_ref,
                          tot_ref, cnt_ref, dist_ref):
    i = pl.program_id(0)
    e_row = erow_ref[...]
    lab_r = labr_ref[...]

    def pass1(j, hn):
        off = j * _C
        w = eallt_ref[:, pl.ds(off, _C)]
        d = jnp.dot(e_row, w, preferred_element_type=jnp.float32)
        d = jnp.maximum(d, 0.0)
        same = lab_r == labc_ref[0:1, pl.ds(off, _C)]
        dist_ref[:, pl.ds(off, _C)] = jnp.where(same, d, -jnp.inf)
        neg = jnp.where(same, jnp.inf, d)
        return jnp.minimum(hn, jnp.min(neg, axis=1, keepdims=True))

    hn0 = jnp.full((_R, 1), jnp.inf, dtype=jnp.float32)
    hn = jax.lax.fori_loop(0, nc, pass1, hn0)

    rr = jax.lax.broadcasted_iota(jnp.int32, (_R, _R), 0)
    cc = jax.lax.broadcasted_iota(jnp.int32, (_R, _R), 1)
    blk = dist_ref[:, pl.ds(i * _R, _R)]
    dist_ref[:, pl.ds(i * _R, _R)] = jnp.where(rr == cc, -jnp.inf, blk)

    def pass2(j, carry):
        off = j * _C
        d = dist_ref[:, pl.ds(off, _C)]
        g = d - hn
        p = jnp.maximum(g, 0.0)
        ind = jnp.where(p > 0.0, 1.0, 0.0)
        tot_ref[0, :, pl.ds(off, _C)] = jnp.sum(p + ind, axis=0, keepdims=True)
        cnt_ref[0, :, pl.ds(off, _C)] = jnp.sum(ind, axis=0, keepdims=True)
        return carry

    jax.lax.fori_loop(0, nc, pass2, 0)


def kernel(embeddings, labels):
    e = embeddings.astype(jnp.float32)
    B, D = e.shape
    lab = labels.astype(jnp.int32)
    nb = B // _R
    nc = B // _C

    sq = jnp.sum(e * e, axis=1)
    hi = sq.astype(jnp.bfloat16).astype(jnp.float32)
    lo = sq - hi
    one = jnp.ones((B, 1), jnp.float32)
    zed = jnp.zeros((B, 4), jnp.float32)
    erow_aug = jnp.concatenate(
        [e, one, one, hi[:, None], lo[:, None], zed], axis=1)
    eallt_aug = jnp.concatenate(
        [(-2.0 * e).T, hi[None, :], lo[None, :], one.T, one.T, zed.T],
        axis=0)
    labr = lab.reshape(B, 1)
    labc = lab.reshape(1, B)

    tot_parts, cnt_parts = pl.pallas_call(
        functools.partial(_triplet_block_kernel, nc),
        grid=(nb,),
        in_specs=[
            pl.BlockSpec((_R, _KAUG), lambda i: (i, 0)),
            pl.BlockSpec((_KAUG, B), lambda i: (0, 0)),
            pl.BlockSpec((_R, 1), lambda i: (i, 0)),
            pl.BlockSpec((1, B), lambda i: (0, 0)),
        ],
        out_specs=[
            pl.BlockSpec((1, 1, B), lambda i: (i, 0, 0)),
            pl.BlockSpec((1, 1, B), lambda i: (i, 0, 0)),
        ],
        out_shape=[
            jax.ShapeDtypeStruct((nb, 1, B), jnp.float32),
            jax.ShapeDtypeStruct((nb, 1, B), jnp.float32),
        ],
        scratch_shapes=[pltpu.VMEM((_R, B), jnp.float32)],
        compiler_params=pltpu.CompilerParams(
            dimension_semantics=("parallel",),
            vmem_limit_bytes=48 * 1024 * 1024,
        ),
        name="triplet_mining",
    )(erow_aug, eallt_aug, labr, labc)

    count = jnp.sum(cnt_parts.astype(jnp.int32))
    total = jnp.sum(tot_parts)
    loss = total / jnp.maximum(count, 1).astype(jnp.float32)
    return loss, count


# speedup vs baseline: 1.5050x; 1.5050x over previous
import functools

import jax
import jax.numpy as jnp
from jax.experimental import pallas as pl
from jax.experimental.pallas import tpu as pltpu

_MARGIN = 1.0
_R = 256
_C = 1024
_KAUG = 136


def _triplet_block_kernel(nc, erow_ref, eallt_ref, labr_ref, labc_ref,
                          tot_ref, cnt_ref, dist_ref):
    i = pl.program_id(0)
    e_row = erow_ref[...]
    lab_r = labr_ref[...]

    def pass1(j, hn):
        off = j * _C
        w = eallt_ref[:, pl.ds(off, _C)]
        d = jnp.dot(e_row, w, preferred_element_type=jnp.float32)
        d = jnp.maximum(d, 0.0)
        same = lab_r == labc_ref[0:1, pl.ds(off, _C)]
        dist_ref[:, pl.ds(off, _C)] = jnp.where(same, d, -jnp.inf)
        neg = jnp.where(same, jnp.inf, d)
        return jnp.minimum(hn, jnp.min(neg, axis=1, keepdims=True))

    hn0 = jnp.full((_R, 1), jnp.inf, dtype=jnp.float32)
    hn = jax.lax.fori_loop(0, nc, pass1, hn0)

    rr = jax.lax.broadcasted_iota(jnp.int32, (_R, _R), 0)
    cc = jax.lax.broadcasted_iota(jnp.int32, (_R, _R), 1)
    blk = dist_ref[:, pl.ds(i * _R, _R)]
    dist_ref[:, pl.ds(i * _R, _R)] = jnp.where(rr == cc, -jnp.inf, blk)

    def pass2(j, carry):
        off = j * _C
        d = dist_ref[:, pl.ds(off, _C)]
        g = d - hn
        p = jnp.maximum(g, 0.0)
        ind = jnp.where(p > 0.0, 1.0, 0.0)
        tot_ref[0, :, pl.ds(off, _C)] = jnp.sum(p + ind, axis=0, keepdims=True)
        cnt_ref[0, :, pl.ds(off, _C)] = jnp.sum(ind, axis=0, keepdims=True)
        return carry

    jax.lax.fori_loop(0, nc, pass2, 0)


def kernel(embeddings, labels):
    e = embeddings.astype(jnp.float32)
    B, D = e.shape
    lab = labels.astype(jnp.int32)
    nb = B // _R
    nc = B // _C

    sq = jnp.sum(e * e, axis=1)
    hi = sq.astype(jnp.bfloat16).astype(jnp.float32)
    lo = sq - hi
    one = jnp.ones((B, 1), jnp.float32)
    zed = jnp.zeros((B, 4), jnp.float32)
    erow_aug = jnp.concatenate(
        [e, one, one, hi[:, None], lo[:, None], zed], axis=1)
    eallt_aug = jnp.concatenate(
        [(-2.0 * e).T, hi[None, :], lo[None, :], one.T, one.T, zed.T],
        axis=0)
    labr = lab.reshape(B, 1)
    labc = lab.reshape(1, B)

    tot_parts, cnt_parts = pl.pallas_call(
        functools.partial(_triplet_block_kernel, nc),
        grid=(nb,),
        in_specs=[
            pl.BlockSpec((_R, _KAUG), lambda i: (i, 0)),
            pl.BlockSpec((_KAUG, B), lambda i: (0, 0)),
            pl.BlockSpec((_R, 1), lambda i: (i, 0)),
            pl.BlockSpec((1, B), lambda i: (0, 0)),
        ],
        out_specs=[
            pl.BlockSpec((1, 1, B), lambda i: (i, 0, 0)),
            pl.BlockSpec((1, 1, B), lambda i: (i, 0, 0)),
        ],
        out_shape=[
            jax.ShapeDtypeStruct((nb, 1, B), jnp.float32),
            jax.ShapeDtypeStruct((nb, 1, B), jnp.float32),
        ],
        scratch_shapes=[pltpu.VMEM((_R, B), jnp.float32)],
        compiler_params=pltpu.CompilerParams(
            dimension_semantics=("parallel",),
            vmem_limit_bytes=48 * 1024 * 1024,
        ),
        name="triplet_mining",
    )(erow_aug, eallt_aug, labr, labc)

    count = jnp.sum(cnt_parts.astype(jnp.int32))
    total = jnp.sum(tot_parts)
    loss = total / jnp.maximum(count, 1).astype(jnp.float32)
    return loss, count


# speedup vs baseline: 1.6924x; 1.1245x over previous
import functools

import jax
import jax.numpy as jnp
from jax.experimental import pallas as pl
from jax.experimental.pallas import tpu as pltpu

_MARGIN = 1.0
_R = 256
_C = 2048
_KAUG = 136


def _triplet_block_kernel(nc, erow_ref, eallt_ref, labr_ref, labc_ref,
                          tot_ref, cnt_ref, dist_ref):
    i = pl.program_id(0)
    e_row = erow_ref[...]
    lab_r = labr_ref[...]

    def pass1(j, hn):
        off = j * _C
        w = eallt_ref[:, pl.ds(off, _C)]
        d = jnp.dot(e_row, w, preferred_element_type=jnp.float32)
        d = jnp.maximum(d, 0.0)
        same = lab_r == labc_ref[0:1, pl.ds(off, _C)]
        dist_ref[:, pl.ds(off, _C)] = jnp.where(same, d, -jnp.inf)
        neg = jnp.where(same, jnp.inf, d)
        return jnp.minimum(hn, jnp.min(neg, axis=1, keepdims=True))

    hn0 = jnp.full((_R, 1), jnp.inf, dtype=jnp.float32)
    hn = jax.lax.fori_loop(0, nc, pass1, hn0)

    rr = jax.lax.broadcasted_iota(jnp.int32, (_R, _R), 0)
    cc = jax.lax.broadcasted_iota(jnp.int32, (_R, _R), 1)
    blk = dist_ref[:, pl.ds(i * _R, _R)]
    dist_ref[:, pl.ds(i * _R, _R)] = jnp.where(rr == cc, -jnp.inf, blk)

    def pass2(j, carry):
        off = j * _C
        d = dist_ref[:, pl.ds(off, _C)]
        g = d - hn
        p = jnp.maximum(g, 0.0)
        ind = jnp.where(p > 0.0, 1.0, 0.0)
        tot_ref[0, :, pl.ds(off, _C)] = jnp.sum(p + ind, axis=0, keepdims=True)
        cnt_ref[0, :, pl.ds(off, _C)] = jnp.sum(ind, axis=0, keepdims=True)
        return carry

    jax.lax.fori_loop(0, nc, pass2, 0)


def kernel(embeddings, labels):
    e = embeddings.astype(jnp.float32)
    B, D = e.shape
    lab = labels.astype(jnp.int32)
    nb = B // _R
    nc = B // _C

    sq = jnp.sum(e * e, axis=1)
    hi = sq.astype(jnp.bfloat16).astype(jnp.float32)
    lo = sq - hi
    one = jnp.ones((B, 1), jnp.float32)
    zed = jnp.zeros((B, 4), jnp.float32)
    erow_aug = jnp.concatenate(
        [e, one, one, hi[:, None], lo[:, None], zed], axis=1)
    eallt_aug = jnp.concatenate(
        [(-2.0 * e).T, hi[None, :], lo[None, :], one.T, one.T, zed.T],
        axis=0)
    labr = lab.reshape(B, 1)
    labc = lab.reshape(1, B)

    tot_parts, cnt_parts = pl.pallas_call(
        functools.partial(_triplet_block_kernel, nc),
        grid=(nb,),
        in_specs=[
            pl.BlockSpec((_R, _KAUG), lambda i: (i, 0)),
            pl.BlockSpec((_KAUG, B), lambda i: (0, 0)),
            pl.BlockSpec((_R, 1), lambda i: (i, 0)),
            pl.BlockSpec((1, B), lambda i: (0, 0)),
        ],
        out_specs=[
            pl.BlockSpec((1, 1, B), lambda i: (i, 0, 0)),
            pl.BlockSpec((1, 1, B), lambda i: (i, 0, 0)),
        ],
        out_shape=[
            jax.ShapeDtypeStruct((nb, 1, B), jnp.float32),
            jax.ShapeDtypeStruct((nb, 1, B), jnp.float32),
        ],
        scratch_shapes=[pltpu.VMEM((_R, B), jnp.float32)],
        compiler_params=pltpu.CompilerParams(
            dimension_semantics=("parallel",),
            vmem_limit_bytes=48 * 1024 * 1024,
        ),
        name="triplet_mining",
    )(erow_aug, eallt_aug, labr, labc)

    count = jnp.sum(cnt_parts.astype(jnp.int32))
    total = jnp.sum(tot_parts)
    loss = total / jnp.maximum(count, 1).astype(jnp.float32)
    return loss, count


# speedup vs baseline: 1.7618x; 1.0410x over previous
import functools

import jax
import jax.numpy as jnp
from jax.experimental import pallas as pl
from jax.experimental.pallas import tpu as pltpu

_MARGIN = 1.0
_R = 256
_C = 4096
_KAUG = 136


def _triplet_block_kernel(nc, erow_ref, eallt_ref, labr_ref, labc_ref,
                          tot_ref, cnt_ref, dist_ref):
    i = pl.program_id(0)
    e_row = erow_ref[...]
    lab_r = labr_ref[...]

    def pass1(j, hn):
        off = j * _C
        w = eallt_ref[:, pl.ds(off, _C)]
        d = jnp.dot(e_row, w, preferred_element_type=jnp.float32)
        d = jnp.maximum(d, 0.0)
        same = lab_r == labc_ref[0:1, pl.ds(off, _C)]
        dist_ref[:, pl.ds(off, _C)] = jnp.where(same, d, -jnp.inf)
        neg = jnp.where(same, jnp.inf, d)
        return jnp.minimum(hn, jnp.min(neg, axis=1, keepdims=True))

    hn0 = jnp.full((_R, 1), jnp.inf, dtype=jnp.float32)
    hn = jax.lax.fori_loop(0, nc, pass1, hn0)

    rr = jax.lax.broadcasted_iota(jnp.int32, (_R, _R), 0)
    cc = jax.lax.broadcasted_iota(jnp.int32, (_R, _R), 1)
    blk = dist_ref[:, pl.ds(i * _R, _R)]
    dist_ref[:, pl.ds(i * _R, _R)] = jnp.where(rr == cc, -jnp.inf, blk)

    def pass2(j, carry):
        off = j * _C
        d = dist_ref[:, pl.ds(off, _C)]
        g = d - hn
        p = jnp.maximum(g, 0.0)
        ind = jnp.where(p > 0.0, 1.0, 0.0)
        tot_ref[0, :, pl.ds(off, _C)] = jnp.sum(p + ind, axis=0, keepdims=True)
        cnt_ref[0, :, pl.ds(off, _C)] = jnp.sum(ind, axis=0, keepdims=True)
        return carry

    jax.lax.fori_loop(0, nc, pass2, 0)


def kernel(embeddings, labels):
    e = embeddings.astype(jnp.float32)
    B, D = e.shape
    lab = labels.astype(jnp.int32)
    nb = B // _R
    nc = B // _C

    sq = jnp.sum(e * e, axis=1)
    hi = sq.astype(jnp.bfloat16).astype(jnp.float32)
    lo = sq - hi
    one = jnp.ones((B, 1), jnp.float32)
    zed = jnp.zeros((B, 4), jnp.float32)
    erow_aug = jnp.concatenate(
        [e, one, one, hi[:, None], lo[:, None], zed], axis=1)
    eallt_aug = jnp.concatenate(
        [(-2.0 * e).T, hi[None, :], lo[None, :], one.T, one.T, zed.T],
        axis=0)
    labr = lab.reshape(B, 1)
    labc = lab.reshape(1, B)

    tot_parts, cnt_parts = pl.pallas_call(
        functools.partial(_triplet_block_kernel, nc),
        grid=(nb,),
        in_specs=[
            pl.BlockSpec((_R, _KAUG), lambda i: (i, 0)),
            pl.BlockSpec((_KAUG, B), lambda i: (0, 0)),
            pl.BlockSpec((_R, 1), lambda i: (i, 0)),
            pl.BlockSpec((1, B), lambda i: (0, 0)),
        ],
        out_specs=[
            pl.BlockSpec((1, 1, B), lambda i: (i, 0, 0)),
            pl.BlockSpec((1, 1, B), lambda i: (i, 0, 0)),
        ],
        out_shape=[
            jax.ShapeDtypeStruct((nb, 1, B), jnp.float32),
            jax.ShapeDtypeStruct((nb, 1, B), jnp.float32),
        ],
        scratch_shapes=[pltpu.VMEM((_R, B), jnp.float32)],
        compiler_params=pltpu.CompilerParams(
            dimension_semantics=("parallel",),
            vmem_limit_bytes=48 * 1024 * 1024,
        ),
        name="triplet_mining",
    )(erow_aug, eallt_aug, labr, labc)

    count = jnp.sum(cnt_parts.astype(jnp.int32))
    total = jnp.sum(tot_parts)
    loss = total / jnp.maximum(count, 1).astype(jnp.float32)
    return loss, count


# speedup vs baseline: 1.8279x; 1.0375x over previous
import functools

import jax
import jax.numpy as jnp
from jax.experimental import pallas as pl
from jax.experimental.pallas import tpu as pltpu

_MARGIN = 1.0
_R = 256
_C = 8192
_KAUG = 136


def _triplet_block_kernel(nc, erow_ref, eallt_ref, labr_ref, labc_ref,
                          tot_ref, cnt_ref, dist_ref):
    i = pl.program_id(0)
    e_row = erow_ref[...]
    lab_r = labr_ref[...]

    def pass1(j, hn):
        off = j * _C
        w = eallt_ref[:, pl.ds(off, _C)]
        d = jnp.dot(e_row, w, preferred_element_type=jnp.float32)
        d = jnp.maximum(d, 0.0)
        same = lab_r == labc_ref[0:1, pl.ds(off, _C)]
        dist_ref[:, pl.ds(off, _C)] = jnp.where(same, d, -jnp.inf)
        neg = jnp.where(same, jnp.inf, d)
        return jnp.minimum(hn, jnp.min(neg, axis=1, keepdims=True))

    hn0 = jnp.full((_R, 1), jnp.inf, dtype=jnp.float32)
    hn = jax.lax.fori_loop(0, nc, pass1, hn0)

    rr = jax.lax.broadcasted_iota(jnp.int32, (_R, _R), 0)
    cc = jax.lax.broadcasted_iota(jnp.int32, (_R, _R), 1)
    blk = dist_ref[:, pl.ds(i * _R, _R)]
    dist_ref[:, pl.ds(i * _R, _R)] = jnp.where(rr == cc, -jnp.inf, blk)

    def pass2(j, carry):
        off = j * _C
        d = dist_ref[:, pl.ds(off, _C)]
        g = d - hn
        p = jnp.maximum(g, 0.0)
        ind = jnp.where(p > 0.0, 1.0, 0.0)
        tot_ref[0, :, pl.ds(off, _C)] = jnp.sum(p + ind, axis=0, keepdims=True)
        cnt_ref[0, :, pl.ds(off, _C)] = jnp.sum(ind, axis=0, keepdims=True)
        return carry

    jax.lax.fori_loop(0, nc, pass2, 0)


def kernel(embeddings, labels):
    e = embeddings.astype(jnp.float32)
    B, D = e.shape
    lab = labels.astype(jnp.int32)
    nb = B // _R
    nc = B // _C

    sq = jnp.sum(e * e, axis=1)
    hi = sq.astype(jnp.bfloat16).astype(jnp.float32)
    lo = sq - hi
    one = jnp.ones((B, 1), jnp.float32)
    zed = jnp.zeros((B, 4), jnp.float32)
    erow_aug = jnp.concatenate(
        [e, one, one, hi[:, None], lo[:, None], zed], axis=1)
    eallt_aug = jnp.concatenate(
        [(-2.0 * e).T, hi[None, :], lo[None, :], one.T, one.T, zed.T],
        axis=0)
    labr = lab.reshape(B, 1)
    labc = lab.reshape(1, B)

    tot_parts, cnt_parts = pl.pallas_call(
        functools.partial(_triplet_block_kernel, nc),
        grid=(nb,),
        in_specs=[
            pl.BlockSpec((_R, _KAUG), lambda i: (i, 0)),
            pl.BlockSpec((_KAUG, B), lambda i: (0, 0)),
            pl.BlockSpec((_R, 1), lambda i: (i, 0)),
            pl.BlockSpec((1, B), lambda i: (0, 0)),
        ],
        out_specs=[
            pl.BlockSpec((1, 1, B), lambda i: (i, 0, 0)),
            pl.BlockSpec((1, 1, B), lambda i: (i, 0, 0)),
        ],
        out_shape=[
            jax.ShapeDtypeStruct((nb, 1, B), jnp.float32),
            jax.ShapeDtypeStruct((nb, 1, B), jnp.float32),
        ],
        scratch_shapes=[pltpu.VMEM((_R, B), jnp.float32)],
        compiler_params=pltpu.CompilerParams(
            dimension_semantics=("parallel",),
            vmem_limit_bytes=48 * 1024 * 1024,
        ),
        name="triplet_mining",
    )(erow_aug, eallt_aug, labr, labc)

    count = jnp.sum(cnt_parts.astype(jnp.int32))
    total = jnp.sum(tot_parts)
    loss = total / jnp.maximum(count, 1).astype(jnp.float32)
    return loss, count


# speedup vs baseline: 2.0727x; 1.1339x over previous
import jax
import jax.numpy as jnp
from jax.experimental import pallas as pl
from jax.experimental.pallas import tpu as pltpu

_MARGIN = 1.0
_R = 256
_KAUG = 136


def _triplet_block_kernel(eall_ref, erow_ref, labcand_ref, labanc_ref,
                          tot_ref, cnt_ref, dist_ref):
    i = pl.program_id(0)

    rhs = erow_ref[...]
    d2 = jax.lax.dot_general(
        eall_ref[...], rhs, (((1,), (1,)), ((), ())),
        preferred_element_type=jnp.float32)
    d2 = jnp.maximum(d2, 0.0)
    same = labcand_ref[...] == labanc_ref[0:1, :]
    dist_ref[...] = jnp.where(same, d2, -jnp.inf)
    neg = jnp.where(same, jnp.inf, d2)
    hn = jnp.min(neg, axis=0, keepdims=True)

    rr = jax.lax.broadcasted_iota(jnp.int32, (_R, _R), 0)
    cc = jax.lax.broadcasted_iota(jnp.int32, (_R, _R), 1)
    blk = dist_ref[pl.ds(i * _R, _R), :]
    dist_ref[pl.ds(i * _R, _R), :] = jnp.where(rr == cc, -jnp.inf, blk)

    d = dist_ref[...]
    g = d - hn
    p = jnp.maximum(g, 0.0)
    ind = jnp.where(p > 0.0, 1.0, 0.0)
    tot_ref[0, :, :] = jnp.sum(p + ind, axis=0, keepdims=True)
    cnt_ref[0, :, :] = jnp.sum(ind, axis=0, keepdims=True)


def kernel(embeddings, labels):
    e = embeddings.astype(jnp.float32)
    B, D = e.shape
    lab = labels.astype(jnp.int32)
    nb = B // _R

    sq = jnp.sum(e * e, axis=1)
    hi = sq.astype(jnp.bfloat16).astype(jnp.float32)
    lo = sq - hi
    one = jnp.ones((B, 1), jnp.float32)
    zed = jnp.zeros((B, 4), jnp.float32)
    eaug_a = jnp.concatenate(
        [-2.0 * e, hi[:, None], lo[:, None], one, one, zed], axis=1)
    eaug_b = jnp.concatenate(
        [e, one, one, hi[:, None], lo[:, None], zed], axis=1)
    labcand = lab.reshape(B, 1)
    labanc = lab.reshape(1, B)

    tot_parts, cnt_parts = pl.pallas_call(
        _triplet_block_kernel,
        grid=(nb,),
        in_specs=[
            pl.BlockSpec((B, _KAUG), lambda i: (0, 0)),
            pl.BlockSpec((_R, _KAUG), lambda i: (i, 0)),
            pl.BlockSpec((B, 1), lambda i: (0, 0)),
            pl.BlockSpec((1, _R), lambda i: (0, i)),
        ],
        out_specs=[
            pl.BlockSpec((1, 1, _R), lambda i: (i, 0, 0)),
            pl.BlockSpec((1, 1, _R), lambda i: (i, 0, 0)),
        ],
        out_shape=[
            jax.ShapeDtypeStruct((nb, 1, _R), jnp.float32),
            jax.ShapeDtypeStruct((nb, 1, _R), jnp.float32),
        ],
        scratch_shapes=[pltpu.VMEM((B, _R), jnp.float32)],
        compiler_params=pltpu.CompilerParams(
            dimension_semantics=("parallel",),
            vmem_limit_bytes=48 * 1024 * 1024,
        ),
        name="triplet_mining",
    )(eaug_a, eaug_b, labcand, labanc)

    count = jnp.sum(cnt_parts.astype(jnp.int32))
    total = jnp.sum(tot_parts)
    loss = total / jnp.maximum(count, 1).astype(jnp.float32)
    return loss, count


# speedup vs baseline: 2.1035x; 1.0149x over previous
import jax
import jax.numpy as jnp
from jax.experimental import pallas as pl
from jax.experimental.pallas import tpu as pltpu

_MARGIN = 1.0
_R = 256
_KAUG = 136


def _triplet_block_kernel(eall_ref, erow_ref, labcand_ref, labanc_ref,
                          tot_ref, cnt_ref, dist_ref):
    i = pl.program_id(0)

    rhs = erow_ref[...]
    d2 = jax.lax.dot_general(
        eall_ref[...], rhs, (((1,), (1,)), ((), ())),
        preferred_element_type=jnp.float32)
    same = labcand_ref[...] == labanc_ref[0:1, :]
    dist_ref[...] = jnp.where(same, d2, -jnp.inf)
    neg = jnp.where(same, jnp.inf, d2)
    hn = jnp.maximum(jnp.min(neg, axis=0, keepdims=True), 0.0)

    rr = jax.lax.broadcasted_iota(jnp.int32, (_R, _R), 0)
    cc = jax.lax.broadcasted_iota(jnp.int32, (_R, _R), 1)
    blk = dist_ref[pl.ds(i * _R, _R), :]
    dist_ref[pl.ds(i * _R, _R), :] = jnp.where(rr == cc, -jnp.inf, blk)

    d = dist_ref[...]
    g = d - hn
    p = jnp.maximum(g, 0.0)
    ind = jnp.where(p > 0.0, 1.0, 0.0)
    tot_ref[0, :, :] = jnp.sum(p, axis=0, keepdims=True)
    cnt_ref[0, :, :] = jnp.sum(ind, axis=0, keepdims=True)


def kernel(embeddings, labels):
    e = embeddings.astype(jnp.float32)
    B, D = e.shape
    lab = labels.astype(jnp.int32)
    nb = B // _R

    sq = jnp.sum(e * e, axis=1)
    hi = sq.astype(jnp.bfloat16).astype(jnp.float32)
    lo = sq - hi
    one = jnp.ones((B, 1), jnp.float32)
    zed = jnp.zeros((B, 4), jnp.float32)
    eaug_a = jnp.concatenate(
        [-2.0 * e, hi[:, None], lo[:, None], one, one, zed], axis=1)
    eaug_b = jnp.concatenate(
        [e, one, one, hi[:, None], lo[:, None], zed], axis=1)
    labcand = lab.reshape(B, 1)
    labanc = lab.reshape(1, B)

    tot_parts, cnt_parts = pl.pallas_call(
        _triplet_block_kernel,
        grid=(nb,),
        in_specs=[
            pl.BlockSpec((B, _KAUG), lambda i: (0, 0)),
            pl.BlockSpec((_R, _KAUG), lambda i: (i, 0)),
            pl.BlockSpec((B, 1), lambda i: (0, 0)),
            pl.BlockSpec((1, _R), lambda i: (0, i)),
        ],
        out_specs=[
            pl.BlockSpec((1, 1, _R), lambda i: (i, 0, 0)),
            pl.BlockSpec((1, 1, _R), lambda i: (i, 0, 0)),
        ],
        out_shape=[
            jax.ShapeDtypeStruct((nb, 1, _R), jnp.float32),
            jax.ShapeDtypeStruct((nb, 1, _R), jnp.float32),
        ],
        scratch_shapes=[pltpu.VMEM((B, _R), jnp.float32)],
        compiler_params=pltpu.CompilerParams(
            dimension_semantics=("parallel",),
            vmem_limit_bytes=48 * 1024 * 1024,
        ),
        name="triplet_mining",
    )(eaug_a, eaug_b, labcand, labanc)

    count = jnp.sum(cnt_parts.astype(jnp.int32))
    total = jnp.sum(tot_parts) + count.astype(jnp.float32)
    loss = total / jnp.maximum(count, 1).astype(jnp.float32)
    return loss, count
